# baseline (speedup 1.0000x reference)
import jax
import jax.numpy as jnp
from jax import lax
from jax.experimental import pallas as pl
from jax.experimental.pallas import tpu as pltpu
from jax.experimental.pallas import tpu_sc as plsc

N = 10000
D = 128
E = 320000
NC = 2
NS = 16
NW = NC * NS
L = 16
NPAD = 10240
CHK = NPAD // NS
CH = 80
EW = CH * 128
SE = 2048
RPT = NPAD // NW
EPAD = NW * EW
PADIDX = NPAD - 1

_mesh = plsc.VectorSubcoreMesh(
    core_axis_name="c", subcore_axis_name="s", num_cores=NC, num_subcores=NS
)
_sc_params = pltpu.CompilerParams(needs_layout_passes=False)


def _combine_and_store(local_v, stage, buf16, sum_v, out_slice, sid):
    pltpu.sync_copy(local_v, stage.at[sid])
    plsc.subcore_barrier()
    pltpu.sync_copy(stage.at[:, pl.ds(sid * CHK, CHK)], buf16)

    @pl.loop(0, CHK // L)
    def _reduce(i):
        acc = buf16[0, pl.ds(i * L, L)]
        for k in range(1, NS):
            acc = acc + buf16[k, pl.ds(i * L, L)]
        sum_v[pl.ds(i * L, L)] = acc

    pltpu.sync_copy(sum_v, out_slice)
    plsc.subcore_barrier()


def _deg_body(ep, deg, src_v, dst_v, dgo_v, dgi_v, sum_v, buf16, stage):
    cid = lax.axis_index("c")
    sid = lax.axis_index("s")
    wid = cid * NS + sid
    pltpu.sync_copy(ep.at[0, wid], src_v)
    pltpu.sync_copy(ep.at[1, wid], dst_v)
    zero16 = jnp.zeros((L,), jnp.int32)

    @pl.loop(0, NPAD // L)
    def _zero(i):
        dgo_v[pl.ds(i * L, L)] = zero16
        dgi_v[pl.ds(i * L, L)] = zero16

    ones16 = jnp.ones((L,), jnp.int32)

    @pl.loop(0, CH)
    def _count(j):
        for k in range(8):
            s = src_v[j, pl.ds(k * L, L)]
            d = dst_v[j, pl.ds(k * L, L)]
            plsc.addupdate_scatter(dgo_v, [s], ones16)
            plsc.addupdate_scatter(dgi_v, [d], ones16)

    for t, dv in ((0, dgo_v), (1, dgi_v)):
        _combine_and_store(dv, stage, buf16, sum_v,
                           deg.at[t, cid, pl.ds(sid * CHK, CHK)], sid)


_deg_call = pl.kernel(
    _deg_body,
    out_type=jax.ShapeDtypeStruct((2, NC, NPAD), jnp.int32),
    mesh=_mesh,
    scratch_types=[
        pltpu.VMEM((CH, 128), jnp.int32),
        pltpu.VMEM((CH, 128), jnp.int32),
        pltpu.VMEM((NPAD,), jnp.int32),
        pltpu.VMEM((NPAD,), jnp.int32),
        pltpu.VMEM((CHK,), jnp.int32),
        pltpu.VMEM((NS, CHK), jnp.int32),
        pltpu.VMEM_SHARED((NS, NPAD), jnp.int32),
    ],
    compiler_params=_sc_params,
)


def _agg_body(ef, xs, a_out, st_s0, st_d0, st_s1, st_d1, csrc, cdst,
              gb0, gb1, acc_v, lsem0, lsem1, gsem0, gsem1):
    cid = lax.axis_index("c")
    sid = lax.axis_index("s")
    w = cid * NS + sid
    zero16f = jnp.zeros((L,), jnp.float32)
    zero16i = jnp.zeros((L,), jnp.int32)
    iota16 = lax.iota(jnp.int32, L)
    NSTRIP = EPAD // SE

    @pl.loop(0, RPT)
    def _za(r):
        for k in range(8):
            acc_v[r, pl.ds(k * L, L)] = zero16f

    @pl.loop(0, (SE + L) // L)
    def _zl(i):
        csrc[pl.ds(i * L, L)] = zero16i
        cdst[pl.ds(i * L, L)] = zero16i

    def accum_chunk(k, gb, cnt):
        pre = []
        for grp in range(4):
            base = k * 64 + grp * 16
            dl = cdst[pl.ds(base, L)]
            valid = (base + iota16) < cnt
            rows = iota16 + grp * 16
            pre.append((dl, valid, rows))

        @pl.loop(0, 8)
        def _cols(cg):
            for cc in range(16):
                cvec = zero16i + (cg * 16 + cc)
                for dl, valid, rows in pre:
                    v = plsc.load_gather(gb, [rows, cvec], mask=valid)
                    plsc.addupdate_scatter(acc_v, [dl, cvec], v, mask=valid)

    def do_strip(t, sbuf, dbuf, lsem, last):
        pltpu.make_async_copy(ef.at[0, pl.ds(t * SE, SE)], sbuf, lsem).wait()
        pltpu.make_async_copy(ef.at[1, pl.ds(t * SE, SE)], dbuf, lsem).wait()

        def scan_body(i, cnt):
            s = sbuf[pl.ds(i * L, L)]
            d = dbuf[pl.ds(i * L, L)]
            b = (d * 6554) >> 21
            m = b == w
            dl = d - b * RPT
            plsc.store_compressed(csrc.at[pl.ds(cnt, L)], s, mask=m)
            plsc.store_compressed(cdst.at[pl.ds(cnt, L)], dl, mask=m)
            return cnt + jnp.sum(m.astype(jnp.int32), axis=0)

        cnt = pl.loop(0, SE // L, init_carry=jnp.int32(0))(scan_body)

        if not last:
            pltpu.async_copy(ef.at[0, pl.ds((t + 2) * SE, SE)], sbuf, lsem)
            pltpu.async_copy(ef.at[1, pl.ds((t + 2) * SE, SE)], dbuf, lsem)

        nch = (cnt + 63) >> 6

        @pl.when(cnt > 0)
        def _():
            pltpu.async_copy(xs.at[csrc.at[pl.ds(0, 64)]], gb0, gsem0)

        @pl.loop(0, nch, step=2)
        def _chunks(k):
            pltpu.make_async_copy(xs.at[csrc.at[pl.ds(k * 64, 64)]], gb0,
                                  gsem0).wait()

            @pl.when(k + 1 < nch)
            def _():
                pltpu.async_copy(xs.at[csrc.at[pl.ds((k + 1) * 64, 64)]],
                                 gb1, gsem1)

            accum_chunk(k, gb0, cnt)

            @pl.when(k + 1 < nch)
            def _():
                pltpu.make_async_copy(xs.at[csrc.at[pl.ds((k + 1) * 64, 64)]],
                                      gb1, gsem1).wait()

                @pl.when(k + 2 < nch)
                def _():
                    pltpu.async_copy(xs.at[csrc.at[pl.ds((k + 2) * 64, 64)]],
                                     gb0, gsem0)

                accum_chunk(k + 1, gb1, cnt)

    pltpu.async_copy(ef.at[0, pl.ds(0, SE)], st_s0, lsem0)
    pltpu.async_copy(ef.at[1, pl.ds(0, SE)], st_d0, lsem0)
    pltpu.async_copy(ef.at[0, pl.ds(SE, SE)], st_s1, lsem1)
    pltpu.async_copy(ef.at[1, pl.ds(SE, SE)], st_d1, lsem1)

    @pl.loop(0, NSTRIP // 2 - 1)
    def _strips(t2):
        do_strip(t2 * 2, st_s0, st_d0, lsem0, False)
        do_strip(t2 * 2 + 1, st_s1, st_d1, lsem1, False)

    do_strip(NSTRIP - 2, st_s0, st_d0, lsem0, True)
    do_strip(NSTRIP - 1, st_s1, st_d1, lsem1, True)

    pltpu.sync_copy(acc_v, a_out.at[pl.ds(w * RPT, RPT)])


_agg_call = pl.kernel(
    _agg_body,
    out_type=jax.ShapeDtypeStruct((NPAD, D), jnp.float32),
    mesh=_mesh,
    scratch_types=[
        pltpu.VMEM((SE,), jnp.int32),
        pltpu.VMEM((SE,), jnp.int32),
        pltpu.VMEM((SE,), jnp.int32),
        pltpu.VMEM((SE,), jnp.int32),
        pltpu.VMEM((SE + L,), jnp.int32),
        pltpu.VMEM((SE + L,), jnp.int32),
        pltpu.VMEM((64, D), jnp.float32),
        pltpu.VMEM((64, D), jnp.float32),
        pltpu.VMEM((RPT, D), jnp.float32),
        pltpu.SemaphoreType.DMA,
        pltpu.SemaphoreType.DMA,
        pltpu.SemaphoreType.DMA,
        pltpu.SemaphoreType.DMA,
    ],
    compiler_params=_sc_params,
)


def _cvec_body(ep, nd, c_out, src_v, dst_v, nd_v, c_v, sum_v, buf16, stage):
    cid = lax.axis_index("c")
    sid = lax.axis_index("s")
    wid = cid * NS + sid
    pltpu.sync_copy(ep.at[0, wid], src_v)
    pltpu.sync_copy(ep.at[1, wid], dst_v)
    pltpu.sync_copy(nd, nd_v)
    zero16 = jnp.zeros((L,), jnp.float32)

    @pl.loop(0, NPAD // L)
    def _zero(i):
        c_v[pl.ds(i * L, L)] = zero16

    @pl.loop(0, CH)
    def _accum(j):
        for k in range(8):
            s = src_v[j, pl.ds(k * L, L)]
            d = dst_v[j, pl.ds(k * L, L)]
            nv = plsc.load_gather(nd_v, [d])
            plsc.addupdate_scatter(c_v, [s], nv)

    _combine_and_store(c_v, stage, buf16, sum_v,
                       c_out.at[cid, pl.ds(sid * CHK, CHK)], sid)


_cvec_call = pl.kernel(
    _cvec_body,
    out_type=jax.ShapeDtypeStruct((NC, NPAD), jnp.float32),
    mesh=_mesh,
    scratch_types=[
        pltpu.VMEM((CH, 128), jnp.int32),
        pltpu.VMEM((CH, 128), jnp.int32),
        pltpu.VMEM((NPAD,), jnp.float32),
        pltpu.VMEM((NPAD,), jnp.float32),
        pltpu.VMEM((CHK,), jnp.float32),
        pltpu.VMEM((NS, CHK), jnp.float32),
        pltpu.VMEM_SHARED((NS, NPAD), jnp.float32),
    ],
    compiler_params=_sc_params,
)


def _prep_body(degp_ref, x_ref, w1_ref, xws_ref, ns_ref, nd_ref):
    d_out = (degp_ref[0, 0] + degp_ref[0, 1]).astype(jnp.float32)
    d_in = (degp_ref[1, 0] + degp_ref[1, 1]).astype(jnp.float32)
    ns = 1.0 / jnp.sqrt(jnp.maximum(d_out, 1.0))
    nd = 1.0 / jnp.sqrt(jnp.maximum(d_in, 1.0))
    ns_ref[...] = ns
    nd_ref[...] = nd
    xw = jnp.dot(x_ref[...], w1_ref[...], preferred_element_type=jnp.float32)
    xws_ref[...] = xw * ns[:N]


_prep_call = pl.pallas_call(
    _prep_body,
    out_shape=(
        jax.ShapeDtypeStruct((N, D), jnp.float32),
        jax.ShapeDtypeStruct((NPAD, 1), jnp.float32),
        jax.ShapeDtypeStruct((NPAD, 1), jnp.float32),
    ),
)


def _fin_body(ap_ref, cp_ref, ns_ref, nd_ref, b1_ref, w2_ref, b2_ref,
              o_ref):
    a = ap_ref[...]
    csum = cp_ref[0] + cp_ref[1]
    h = jnp.maximum(a * nd_ref[...] + b1_ref[...], 0.0)
    rows = lax.broadcasted_iota(jnp.int32, (NPAD, 1), 0)
    w = jnp.where(rows < N, ns_ref[...] * csum, 0.0) * (1.0 / N)
    srow = jnp.sum(h * w, axis=0, keepdims=True)
    o_ref[...] = jnp.sum(srow * w2_ref[...], axis=1, keepdims=True) \
        + b2_ref[...]


_fin_call = pl.pallas_call(
    _fin_body,
    out_shape=jax.ShapeDtypeStruct((1, 1), jnp.float32),
)


def kernel(x, W1, b1, W2, b2, edge_index):
    pad = jnp.full((2, EPAD - E), PADIDX, dtype=jnp.int32)
    ep = jnp.concatenate([edge_index.astype(jnp.int32), pad], axis=1)
    ep = ep.reshape(2, NW, CH, 128)

    deg = _deg_call(ep)
    xws, ns, nd = _prep_call(deg.reshape(2, NC, NPAD, 1), x, W1)
    a_p = _agg_call(ep.reshape(2, EPAD),
                    jnp.pad(xws, ((0, NPAD - N), (0, 0))))
    c_p = _cvec_call(ep, nd.reshape(NPAD))
    out = _fin_call(a_p, c_p.reshape(NC, NPAD, 1), ns, nd,
                    b1.reshape(1, D), W2.reshape(1, D), b2.reshape(1, 1))
    return out.reshape(1)


# speedup vs baseline: 2.1718x; 2.1718x over previous
import jax
import jax.numpy as jnp
from jax import lax
from jax.experimental import pallas as pl
from jax.experimental.pallas import tpu as pltpu
from jax.experimental.pallas import tpu_sc as plsc

N = 10000
D = 128
E = 320000
NC = 2
NS = 16
NW = NC * NS
L = 16
NPAD = 10240
CHK = NPAD // NS
CH = 80
EW = CH * 128
SE = 8192
RPT = NPAD // NW
EPAD = NW * EW
PADIDX = NPAD - 1

_mesh = plsc.VectorSubcoreMesh(
    core_axis_name="c", subcore_axis_name="s", num_cores=NC, num_subcores=NS
)
_sc_params = pltpu.CompilerParams(needs_layout_passes=False)


def _combine_and_store(local_v, stage, buf16, sum_v, out_slice, sid):
    pltpu.sync_copy(local_v, stage.at[sid])
    plsc.subcore_barrier()
    pltpu.sync_copy(stage.at[:, pl.ds(sid * CHK, CHK)], buf16)

    @pl.loop(0, CHK // L)
    def _reduce(i):
        acc = buf16[0, pl.ds(i * L, L)]
        for k in range(1, NS):
            acc = acc + buf16[k, pl.ds(i * L, L)]
        sum_v[pl.ds(i * L, L)] = acc

    pltpu.sync_copy(sum_v, out_slice)
    plsc.subcore_barrier()


def _deg_body(ep, deg, src_v, dst_v, dgo_v, dgi_v, sum_v, buf16, stage):
    cid = lax.axis_index("c")
    sid = lax.axis_index("s")
    wid = cid * NS + sid
    pltpu.sync_copy(ep.at[0, wid], src_v)
    pltpu.sync_copy(ep.at[1, wid], dst_v)
    zero16 = jnp.zeros((L,), jnp.int32)

    @pl.loop(0, NPAD // L)
    def _zero(i):
        dgo_v[pl.ds(i * L, L)] = zero16
        dgi_v[pl.ds(i * L, L)] = zero16

    ones16 = jnp.ones((L,), jnp.int32)

    @pl.loop(0, CH)
    def _count(j):
        for k in range(8):
            s = src_v[j, pl.ds(k * L, L)]
            d = dst_v[j, pl.ds(k * L, L)]
            plsc.addupdate_scatter(dgo_v, [s], ones16)
            plsc.addupdate_scatter(dgi_v, [d], ones16)

    for t, dv in ((0, dgo_v), (1, dgi_v)):
        _combine_and_store(dv, stage, buf16, sum_v,
                           deg.at[t, cid, pl.ds(sid * CHK, CHK)], sid)


_deg_call = pl.kernel(
    _deg_body,
    out_type=jax.ShapeDtypeStruct((2, NC, NPAD), jnp.int32),
    mesh=_mesh,
    scratch_types=[
        pltpu.VMEM((CH, 128), jnp.int32),
        pltpu.VMEM((CH, 128), jnp.int32),
        pltpu.VMEM((NPAD,), jnp.int32),
        pltpu.VMEM((NPAD,), jnp.int32),
        pltpu.VMEM((CHK,), jnp.int32),
        pltpu.VMEM((NS, CHK), jnp.int32),
        pltpu.VMEM_SHARED((NS, NPAD), jnp.int32),
    ],
    compiler_params=_sc_params,
)


def _agg_body(ef, xs, a_out, st_s0, st_d0, st_s1, st_d1, csrc, cdst,
              gb0, gb1, acc_v, lsem0, lsem1, gsem0, gsem1):
    cid = lax.axis_index("c")
    sid = lax.axis_index("s")
    w = cid * NS + sid
    zero16f = jnp.zeros((L,), jnp.float32)
    zero16i = jnp.zeros((L,), jnp.int32)
    iota16 = lax.iota(jnp.int32, L)
    NSTRIP = EPAD // SE

    @pl.loop(0, RPT)
    def _za(r):
        for k in range(8):
            acc_v[r, pl.ds(k * L, L)] = zero16f

    @pl.loop(0, (SE + 64) // L)
    def _zl(i):
        csrc[pl.ds(i * L, L)] = zero16i
        cdst[pl.ds(i * L, L)] = zero16i

    def accum_chunk(k, gb):
        pre = []
        for grp in range(4):
            base = k * 64 + grp * 16
            dl = cdst[pl.ds(base, L)]
            rows = iota16 + grp * 16
            pre.append((dl, rows))

        @pl.loop(0, 8)
        def _cols(cg):
            for cc in range(16):
                cvec = zero16i + (cg * 16 + cc)
                for dl, rows in pre:
                    v = plsc.load_gather(gb, [rows, cvec])
                    plsc.addupdate_scatter(acc_v, [dl, cvec], v)

    def do_strip(t, sbuf, dbuf, lsem, last):
        pltpu.make_async_copy(ef.at[0, pl.ds(t * SE, SE)], sbuf, lsem).wait()
        pltpu.make_async_copy(ef.at[1, pl.ds(t * SE, SE)], dbuf, lsem).wait()

        def scan_body(i, cnt):
            s = sbuf[pl.ds(i * L, L)]
            d = dbuf[pl.ds(i * L, L)]
            b = (d * 6554) >> 21
            m = b == w
            dl = d - b * RPT
            plsc.store_compressed(csrc.at[pl.ds(cnt, L)], s, mask=m)
            plsc.store_compressed(cdst.at[pl.ds(cnt, L)], dl, mask=m)
            return cnt + jnp.sum(m.astype(jnp.int32), axis=0)

        cnt = pl.loop(0, SE // L, init_carry=jnp.int32(0))(scan_body)

        if not last:
            pltpu.async_copy(ef.at[0, pl.ds((t + 2) * SE, SE)], sbuf, lsem)
            pltpu.async_copy(ef.at[1, pl.ds((t + 2) * SE, SE)], dbuf, lsem)

        pad_s = zero16i + PADIDX
        for q in range(4):
            pq = cnt + q * L
            csrc[pl.ds(pq, L)] = pad_s
            cdst[pl.ds(pq, L)] = zero16i
        nch = (cnt + 63) >> 6

        pltpu.async_copy(xs.at[csrc.at[pl.ds(0, 64)]], gb0, gsem0)

        @pl.loop(0, nch, step=2)
        def _chunks(k):
            pltpu.make_async_copy(xs.at[csrc.at[pl.ds(k * 64, 64)]], gb0,
                                  gsem0).wait()

            @pl.when(k + 1 < nch)
            def _():
                pltpu.async_copy(xs.at[csrc.at[pl.ds((k + 1) * 64, 64)]],
                                 gb1, gsem1)

            accum_chunk(k, gb0)

            @pl.when(k + 1 < nch)
            def _():
                pltpu.make_async_copy(xs.at[csrc.at[pl.ds((k + 1) * 64, 64)]],
                                      gb1, gsem1).wait()

                @pl.when(k + 2 < nch)
                def _():
                    pltpu.async_copy(xs.at[csrc.at[pl.ds((k + 2) * 64, 64)]],
                                     gb0, gsem0)

                accum_chunk(k + 1, gb1)

    pltpu.async_copy(ef.at[0, pl.ds(0, SE)], st_s0, lsem0)
    pltpu.async_copy(ef.at[1, pl.ds(0, SE)], st_d0, lsem0)
    pltpu.async_copy(ef.at[0, pl.ds(SE, SE)], st_s1, lsem1)
    pltpu.async_copy(ef.at[1, pl.ds(SE, SE)], st_d1, lsem1)

    @pl.loop(0, NSTRIP // 2 - 1)
    def _strips(t2):
        do_strip(t2 * 2, st_s0, st_d0, lsem0, False)
        do_strip(t2 * 2 + 1, st_s1, st_d1, lsem1, False)

    do_strip(NSTRIP - 2, st_s0, st_d0, lsem0, True)
    do_strip(NSTRIP - 1, st_s1, st_d1, lsem1, True)

    pltpu.sync_copy(acc_v, a_out.at[pl.ds(w * RPT, RPT)])


_agg_call = pl.kernel(
    _agg_body,
    out_type=jax.ShapeDtypeStruct((NPAD, D), jnp.float32),
    mesh=_mesh,
    scratch_types=[
        pltpu.VMEM((SE,), jnp.int32),
        pltpu.VMEM((SE,), jnp.int32),
        pltpu.VMEM((SE,), jnp.int32),
        pltpu.VMEM((SE,), jnp.int32),
        pltpu.VMEM((SE + 64,), jnp.int32),
        pltpu.VMEM((SE + 64,), jnp.int32),
        pltpu.VMEM((64, D), jnp.float32),
        pltpu.VMEM((64, D), jnp.float32),
        pltpu.VMEM((RPT, D), jnp.float32),
        pltpu.SemaphoreType.DMA,
        pltpu.SemaphoreType.DMA,
        pltpu.SemaphoreType.DMA,
        pltpu.SemaphoreType.DMA,
    ],
    compiler_params=_sc_params,
)


def _cvec_body(ep, nd, c_out, src_v, dst_v, nd_v, c_v, sum_v, buf16, stage):
    cid = lax.axis_index("c")
    sid = lax.axis_index("s")
    wid = cid * NS + sid
    pltpu.sync_copy(ep.at[0, wid], src_v)
    pltpu.sync_copy(ep.at[1, wid], dst_v)
    pltpu.sync_copy(nd, nd_v)
    zero16 = jnp.zeros((L,), jnp.float32)

    @pl.loop(0, NPAD // L)
    def _zero(i):
        c_v[pl.ds(i * L, L)] = zero16

    @pl.loop(0, CH)
    def _accum(j):
        for k in range(8):
            s = src_v[j, pl.ds(k * L, L)]
            d = dst_v[j, pl.ds(k * L, L)]
            nv = plsc.load_gather(nd_v, [d])
            plsc.addupdate_scatter(c_v, [s], nv)

    _combine_and_store(c_v, stage, buf16, sum_v,
                       c_out.at[cid, pl.ds(sid * CHK, CHK)], sid)


_cvec_call = pl.kernel(
    _cvec_body,
    out_type=jax.ShapeDtypeStruct((NC, NPAD), jnp.float32),
    mesh=_mesh,
    scratch_types=[
        pltpu.VMEM((CH, 128), jnp.int32),
        pltpu.VMEM((CH, 128), jnp.int32),
        pltpu.VMEM((NPAD,), jnp.float32),
        pltpu.VMEM((NPAD,), jnp.float32),
        pltpu.VMEM((CHK,), jnp.float32),
        pltpu.VMEM((NS, CHK), jnp.float32),
        pltpu.VMEM_SHARED((NS, NPAD), jnp.float32),
    ],
    compiler_params=_sc_params,
)


def _prep_body(degp_ref, x_ref, w1_ref, xws_ref, ns_ref, nd_ref):
    d_out = (degp_ref[0, 0] + degp_ref[0, 1]).astype(jnp.float32)
    d_in = (degp_ref[1, 0] + degp_ref[1, 1]).astype(jnp.float32)
    ns = 1.0 / jnp.sqrt(jnp.maximum(d_out, 1.0))
    nd = 1.0 / jnp.sqrt(jnp.maximum(d_in, 1.0))
    ns_ref[...] = ns
    nd_ref[...] = nd
    xw = jnp.dot(x_ref[...], w1_ref[...], preferred_element_type=jnp.float32)
    xws_ref[...] = xw * ns[:N]


_prep_call = pl.pallas_call(
    _prep_body,
    out_shape=(
        jax.ShapeDtypeStruct((N, D), jnp.float32),
        jax.ShapeDtypeStruct((NPAD, 1), jnp.float32),
        jax.ShapeDtypeStruct((NPAD, 1), jnp.float32),
    ),
)


def _fin_body(ap_ref, cp_ref, ns_ref, nd_ref, b1_ref, w2_ref, b2_ref,
              o_ref):
    a = ap_ref[...]
    csum = cp_ref[0] + cp_ref[1]
    h = jnp.maximum(a * nd_ref[...] + b1_ref[...], 0.0)
    rows = lax.broadcasted_iota(jnp.int32, (NPAD, 1), 0)
    w = jnp.where(rows < N, ns_ref[...] * csum, 0.0) * (1.0 / N)
    srow = jnp.sum(h * w, axis=0, keepdims=True)
    o_ref[...] = jnp.sum(srow * w2_ref[...], axis=1, keepdims=True) \
        + b2_ref[...]


_fin_call = pl.pallas_call(
    _fin_body,
    out_shape=jax.ShapeDtypeStruct((1, 1), jnp.float32),
)


def kernel(x, W1, b1, W2, b2, edge_index):
    pad = jnp.full((2, EPAD - E), PADIDX, dtype=jnp.int32)
    ep = jnp.concatenate([edge_index.astype(jnp.int32), pad], axis=1)
    ep = ep.reshape(2, NW, CH, 128)

    deg = _deg_call(ep)
    xws, ns, nd = _prep_call(deg.reshape(2, NC, NPAD, 1), x, W1)
    a_p = _agg_call(ep.reshape(2, EPAD),
                    jnp.pad(xws, ((0, NPAD - N), (0, 0))))
    c_p = _cvec_call(ep, nd.reshape(NPAD))
    out = _fin_call(a_p, c_p.reshape(NC, NPAD, 1), ns, nd,
                    b1.reshape(1, D), W2.reshape(1, D), b2.reshape(1, 1))
    return out.reshape(1)


# speedup vs baseline: 12.4554x; 5.7349x over previous
import jax
import jax.numpy as jnp
from jax import lax
from jax.experimental import pallas as pl
from jax.experimental.pallas import tpu as pltpu
from jax.experimental.pallas import tpu_sc as plsc

N = 10000
D = 128
E = 320000
NC = 2
NS = 16
NW = NC * NS
L = 16
NPAD = 10240
CHK = NPAD // NS
CH = 80
AC = 160
ACH = AC // 4
NBUF = 4
EW = CH * 128
EPAD = NW * EW
PADIDX = NPAD - 1

_mesh = plsc.VectorSubcoreMesh(
    core_axis_name="c", subcore_axis_name="s", num_cores=NC, num_subcores=NS
)
_sc_params = pltpu.CompilerParams(needs_layout_passes=False)


def _combine_and_store(local_v, stage, buf16, sum_v, out_slice, sid):
    pltpu.sync_copy(local_v, stage.at[sid])
    plsc.subcore_barrier()
    pltpu.sync_copy(stage.at[:, pl.ds(sid * CHK, CHK)], buf16)

    @pl.loop(0, CHK // L)
    def _reduce(i):
        acc = buf16[0, pl.ds(i * L, L)]
        for k in range(1, NS):
            acc = acc + buf16[k, pl.ds(i * L, L)]
        sum_v[pl.ds(i * L, L)] = acc

    pltpu.sync_copy(sum_v, out_slice)
    plsc.subcore_barrier()


def _deg_body(ep, deg, src_v, dst_v, dgo_v, dgi_v, sum_v, buf16, stage):
    cid = lax.axis_index("c")
    sid = lax.axis_index("s")
    wid = cid * NS + sid
    pltpu.sync_copy(ep.at[0, wid], src_v)
    pltpu.sync_copy(ep.at[1, wid], dst_v)
    zero16 = jnp.zeros((L,), jnp.int32)

    @pl.loop(0, NPAD // L)
    def _zero(i):
        dgo_v[pl.ds(i * L, L)] = zero16
        dgi_v[pl.ds(i * L, L)] = zero16

    ones16 = jnp.ones((L,), jnp.int32)

    @pl.loop(0, CH)
    def _count(j):
        for k in range(8):
            s = src_v[j, pl.ds(k * L, L)]
            d = dst_v[j, pl.ds(k * L, L)]
            plsc.addupdate_scatter(dgo_v, [s], ones16)
            plsc.addupdate_scatter(dgi_v, [d], ones16)

    for t, dv in ((0, dgo_v), (1, dgi_v)):
        _combine_and_store(dv, stage, buf16, sum_v,
                           deg.at[t, cid, pl.ds(sid * CHK, CHK)], sid)


_deg_call = pl.kernel(
    _deg_body,
    out_type=jax.ShapeDtypeStruct((2, NC, NPAD), jnp.int32),
    mesh=_mesh,
    scratch_types=[
        pltpu.VMEM((CH, 128), jnp.int32),
        pltpu.VMEM((CH, 128), jnp.int32),
        pltpu.VMEM((NPAD,), jnp.int32),
        pltpu.VMEM((NPAD,), jnp.int32),
        pltpu.VMEM((CHK,), jnp.int32),
        pltpu.VMEM((NS, CHK), jnp.int32),
        pltpu.VMEM_SHARED((NS, NPAD), jnp.int32),
    ],
    compiler_params=_sc_params,
)


def _agg_body(ep, xs, a_out, src_v, dst_v, rb0, rb1, rb2, rb3,
              acc_sh, gs0, gs1, gs2, gs3, ss0, ss1, ss2, ss3):
    cid = lax.axis_index("c")
    sid = lax.axis_index("s")
    wid = cid * NS + sid
    rbs = (rb0, rb1, rb2, rb3)
    gsem = (gs0, gs1, gs2, gs3)
    ssem = (ss0, ss1, ss2, ss3)
    zero16 = jnp.zeros((L,), jnp.float32)

    @pl.loop(0, 64)
    def _zero_rb(r):
        for k in range(8):
            rb0[r, pl.ds(k * L, L)] = zero16

    for i in range(CHK // 64):
        pltpu.sync_copy(rb0, acc_sh.at[pl.ds(sid * CHK + i * 64, 64)])
    plsc.subcore_barrier()

    for half in range(4):
        pltpu.sync_copy(ep.at[0, wid, pl.ds(half * ACH, ACH)], src_v)
        pltpu.sync_copy(ep.at[1, wid, pl.ds(half * ACH, ACH)], dst_v)
        for b in range(NBUF):
            pltpu.async_copy(xs.at[src_v.at[b]], rbs[b], gsem[b])

        @pl.loop(0, ACH // NBUF - 1)
        def _main(it):
            g = it * NBUF
            sd = []
            for b in range(NBUF):
                pltpu.make_async_copy(xs.at[src_v.at[g + b]], rbs[b],
                                      gsem[b]).wait()
                sd.append(pltpu.async_copy(rbs[b], acc_sh.at[dst_v.at[g + b]],
                                           ssem[b], add=True))
            for b in range(NBUF):
                sd[b].wait()
                pltpu.async_copy(xs.at[src_v.at[g + NBUF + b]], rbs[b],
                                 gsem[b])

        g = ACH - NBUF
        sd = []
        for b in range(NBUF):
            pltpu.make_async_copy(xs.at[src_v.at[g + b]], rbs[b],
                                  gsem[b]).wait()
            sd.append(pltpu.async_copy(rbs[b], acc_sh.at[dst_v.at[g + b]],
                                       ssem[b], add=True))
        for b in range(NBUF):
            sd[b].wait()

    plsc.subcore_barrier()
    pltpu.sync_copy(acc_sh.at[pl.ds(sid * CHK, CHK)],
                    a_out.at[cid, pl.ds(sid * CHK, CHK)])


_agg_call = pl.kernel(
    _agg_body,
    out_type=jax.ShapeDtypeStruct((NC, NPAD, D), jnp.float32),
    mesh=_mesh,
    scratch_types=[
        pltpu.VMEM((ACH, 64), jnp.int32),
        pltpu.VMEM((ACH, 64), jnp.int32),
        pltpu.VMEM((64, D), jnp.float32),
        pltpu.VMEM((64, D), jnp.float32),
        pltpu.VMEM((64, D), jnp.float32),
        pltpu.VMEM((64, D), jnp.float32),
        pltpu.VMEM_SHARED((NPAD, D), jnp.float32),
        pltpu.SemaphoreType.DMA,
        pltpu.SemaphoreType.DMA,
        pltpu.SemaphoreType.DMA,
        pltpu.SemaphoreType.DMA,
        pltpu.SemaphoreType.DMA,
        pltpu.SemaphoreType.DMA,
        pltpu.SemaphoreType.DMA,
        pltpu.SemaphoreType.DMA,
    ],
    compiler_params=_sc_params,
)


def _cvec_body(ep, nd, c_out, src_v, dst_v, nd_v, c_v, sum_v, buf16, stage):
    cid = lax.axis_index("c")
    sid = lax.axis_index("s")
    wid = cid * NS + sid
    pltpu.sync_copy(ep.at[0, wid], src_v)
    pltpu.sync_copy(ep.at[1, wid], dst_v)
    pltpu.sync_copy(nd, nd_v)
    zero16 = jnp.zeros((L,), jnp.float32)

    @pl.loop(0, NPAD // L)
    def _zero(i):
        c_v[pl.ds(i * L, L)] = zero16

    @pl.loop(0, CH)
    def _accum(j):
        for k in range(8):
            s = src_v[j, pl.ds(k * L, L)]
            d = dst_v[j, pl.ds(k * L, L)]
            nv = plsc.load_gather(nd_v, [d])
            plsc.addupdate_scatter(c_v, [s], nv)

    _combine_and_store(c_v, stage, buf16, sum_v,
                       c_out.at[cid, pl.ds(sid * CHK, CHK)], sid)


_cvec_call = pl.kernel(
    _cvec_body,
    out_type=jax.ShapeDtypeStruct((NC, NPAD), jnp.float32),
    mesh=_mesh,
    scratch_types=[
        pltpu.VMEM((CH, 128), jnp.int32),
        pltpu.VMEM((CH, 128), jnp.int32),
        pltpu.VMEM((NPAD,), jnp.float32),
        pltpu.VMEM((NPAD,), jnp.float32),
        pltpu.VMEM((CHK,), jnp.float32),
        pltpu.VMEM((NS, CHK), jnp.float32),
        pltpu.VMEM_SHARED((NS, NPAD), jnp.float32),
    ],
    compiler_params=_sc_params,
)


def _prep_body(degp_ref, x_ref, w1_ref, xws_ref, ns_ref, nd_ref):
    d_out = (degp_ref[0, 0] + degp_ref[0, 1]).astype(jnp.float32)
    d_in = (degp_ref[1, 0] + degp_ref[1, 1]).astype(jnp.float32)
    ns = 1.0 / jnp.sqrt(jnp.maximum(d_out, 1.0))
    nd = 1.0 / jnp.sqrt(jnp.maximum(d_in, 1.0))
    ns_ref[...] = ns
    nd_ref[...] = nd
    xw = jnp.dot(x_ref[...], w1_ref[...], preferred_element_type=jnp.float32)
    xws_ref[...] = xw * ns[:N]


_prep_call = pl.pallas_call(
    _prep_body,
    out_shape=(
        jax.ShapeDtypeStruct((N, D), jnp.float32),
        jax.ShapeDtypeStruct((NPAD, 1), jnp.float32),
        jax.ShapeDtypeStruct((NPAD, 1), jnp.float32),
    ),
)


def _fin_body(ap_ref, cp_ref, ns_ref, nd_ref, b1_ref, w2_ref, b2_ref,
              o_ref):
    a = ap_ref[0] + ap_ref[1]
    csum = cp_ref[0] + cp_ref[1]
    h = jnp.maximum(a * nd_ref[...] + b1_ref[...], 0.0)
    rows = lax.broadcasted_iota(jnp.int32, (NPAD, 1), 0)
    w = jnp.where(rows < N, ns_ref[...] * csum, 0.0) * (1.0 / N)
    srow = jnp.sum(h * w, axis=0, keepdims=True)
    o_ref[...] = jnp.sum(srow * w2_ref[...], axis=1, keepdims=True) \
        + b2_ref[...]


_fin_call = pl.pallas_call(
    _fin_body,
    out_shape=jax.ShapeDtypeStruct((1, 1), jnp.float32),
)


def kernel(x, W1, b1, W2, b2, edge_index):
    pad = jnp.full((2, EPAD - E), PADIDX, dtype=jnp.int32)
    ep = jnp.concatenate([edge_index.astype(jnp.int32), pad], axis=1)
    ep = ep.reshape(2, NW, CH, 128)

    deg = _deg_call(ep)
    xws, ns, nd = _prep_call(deg.reshape(2, NC, NPAD, 1), x, W1)
    a_p = _agg_call(ep.reshape(2, NW, AC, 64),
                    jnp.pad(xws, ((0, NPAD - N), (0, 0))))
    c_p = _cvec_call(ep, nd.reshape(NPAD))
    out = _fin_call(a_p, c_p.reshape(NC, NPAD, 1), ns, nd,
                    b1.reshape(1, D), W2.reshape(1, D), b2.reshape(1, 1))
    return out.reshape(1)
